# Initial kernel scaffold; baseline (speedup 1.0000x reference)
import jax
import jax.numpy as jnp
from jax.experimental import pallas as pl


def kernel(x, w, gamma, beta):
    raise NotImplementedError("write your pallas kernel here")


# speedup vs baseline: 1.9306x; 1.9306x over previous
import functools

import jax
import jax.numpy as jnp
from jax import lax
from jax.experimental import pallas as pl
from jax.experimental.pallas import tpu as pltpu


def _matmul_stats_kernel(x_ref, w_ref, y_ref, sum_ref, ssq_ref):
    @pl.when(pl.program_id(1) == 0)
    def _():
        sum_ref[...] = jnp.zeros_like(sum_ref)
        ssq_ref[...] = jnp.zeros_like(ssq_ref)

    y = jnp.dot(x_ref[...].astype(jnp.bfloat16), w_ref[...],
                preferred_element_type=jnp.float32)
    sum_ref[...] += jnp.sum(y, axis=0, keepdims=True)[None]
    ssq_ref[...] += jnp.sum(y * y, axis=0, keepdims=True)[None]
    y_ref[...] = y.astype(jnp.bfloat16)


def _apply_kernel(y_ref, scale_ref, bias_ref, o_ref, *, negative_slope):
    z = y_ref[...].astype(jnp.float32) * scale_ref[...] + bias_ref[...]
    if negative_slope is not None:
        z = jnp.maximum(z, negative_slope * z)
    o_ref[...] = z.astype(o_ref.dtype)


def _pick_tile(n):
    for t in (1024, 1000, 800, 640, 512, 500, 400, 256, 250, 200, 128, 125,
              104, 100, 64, 40, 32, 16, 8):
        if t % 8 == 0 and n % t == 0 and (n // t) % 2 == 0:
            return t
    return None


def kernel(x, w, gamma, beta):
    num_group = 32
    eps = 1e-5
    negative_slope = 0.1

    n, din = x.shape
    dout = w.shape[0]
    cg = dout // num_group

    tile_n = _pick_tile(n)
    if tile_n is None:
        tile_n = 1024
        num_tiles = pl.cdiv(n, tile_n)
        num_tiles += num_tiles % 2
        n_pad = num_tiles * tile_n
        x_pad = jnp.pad(x, ((0, n_pad - n), (0, 0)))
    else:
        num_tiles = n // tile_n
        n_pad = n
        x_pad = x
    half = num_tiles // 2

    w_t = jnp.transpose(w).astype(jnp.bfloat16)

    y_bf16, sum_pc, ssq_pc = pl.pallas_call(
        _matmul_stats_kernel,
        out_shape=(jax.ShapeDtypeStruct((n_pad, dout), jnp.bfloat16),
                   jax.ShapeDtypeStruct((2, 1, dout), jnp.float32),
                   jax.ShapeDtypeStruct((2, 1, dout), jnp.float32)),
        grid=(2, half),
        in_specs=[
            pl.BlockSpec((tile_n, din), lambda i, j: (i * half + j, 0)),
            pl.BlockSpec((din, dout), lambda i, j: (0, 0)),
        ],
        out_specs=(
            pl.BlockSpec((tile_n, dout), lambda i, j: (i * half + j, 0)),
            pl.BlockSpec((1, 1, dout), lambda i, j: (i, 0, 0)),
            pl.BlockSpec((1, 1, dout), lambda i, j: (i, 0, 0)),
        ),
        compiler_params=pltpu.CompilerParams(
            dimension_semantics=("parallel", "arbitrary")),
    )(x_pad, w_t)

    count = jnp.float32(n) * cg
    sum_c = jnp.sum(sum_pc, axis=(0, 1))
    ssq_c = jnp.sum(ssq_pc, axis=(0, 1))
    g_sum = jnp.sum(sum_c.reshape(num_group, cg), axis=1)
    g_ssq = jnp.sum(ssq_c.reshape(num_group, cg), axis=1)
    mean_g = g_sum / count
    var_g = jnp.maximum(g_ssq / count - mean_g * mean_g, 0.0)
    inv_g = lax.rsqrt(var_g + eps)
    scale_c = gamma.astype(jnp.float32) * jnp.repeat(inv_g, cg)
    bias_c = beta.astype(jnp.float32) - jnp.repeat(mean_g, cg) * scale_c
    scale_2d = scale_c.reshape(1, dout)
    bias_2d = bias_c.reshape(1, dout)

    apply_fn = functools.partial(_apply_kernel, negative_slope=negative_slope)
    out_pad = pl.pallas_call(
        apply_fn,
        out_shape=jax.ShapeDtypeStruct((n_pad, dout), x.dtype),
        grid=(num_tiles,),
        in_specs=[
            pl.BlockSpec((tile_n, dout), lambda i: (i, 0)),
            pl.BlockSpec((1, dout), lambda i: (0, 0)),
            pl.BlockSpec((1, dout), lambda i: (0, 0)),
        ],
        out_specs=pl.BlockSpec((tile_n, dout), lambda i: (i, 0)),
        compiler_params=pltpu.CompilerParams(
            dimension_semantics=("parallel",)),
    )(y_bf16, scale_2d, bias_2d)

    out = out_pad if n_pad == n else out_pad[:n]
    return jnp.squeeze(out)
